# baseline (speedup 1.0000x reference)
import functools

import jax
import jax.numpy as jnp
from jax import lax
from jax.experimental import pallas as pl
from jax.experimental.pallas import tpu as pltpu
from jax.experimental.pallas import tpu_sc as plsc

_NC = 2
_NS = 16
_NW = _NC * _NS
_L = 16


def _sc_prep_body(m, k, chunk, t_hbm, tsb_hbm, es_hbm,
                  w_hbm, e0_hbm, e1_hbm,
                  tsb_v, t_v, w_v, i0_v, i1_v, e0_v, e1_v, sem):
    wid = lax.axis_index("s") * _NC + lax.axis_index("c")
    base = wid * chunk

    pltpu.sync_copy(t_hbm.at[pl.ds(base, chunk)], t_v)
    pltpu.sync_copy(tsb_hbm, tsb_v)

    lo = tsb_v[pl.ds(0, _L)]
    hi = tsb_v[pl.ds((m - 1) * _L, _L)]
    onev = jnp.full((_L,), 1, jnp.int32)
    topv = jnp.full((_L,), m - 1, jnp.int32)
    epsv = jnp.full((_L,), 1e-12, jnp.float32)
    zeroi = jnp.zeros((_L,), jnp.int32)
    zerof = jnp.zeros((_L,), jnp.float32)
    for c in range(chunk // _L):
        tcv = jnp.minimum(jnp.maximum(t_v[pl.ds(c * _L, _L)], lo), hi)
        idx = zeroi
        for mm in range(m):
            knot = tsb_v[pl.ds(mm * _L, _L)]
            idx = idx + jnp.where(knot < tcv, onev, zeroi)
        idx = jnp.minimum(jnp.maximum(idx, onev), topv)
        i0 = idx - onev
        t0 = zerof
        t1 = zerof
        for mm in range(m):
            mmv = jnp.full((_L,), mm, jnp.int32)
            knot = tsb_v[pl.ds(mm * _L, _L)]
            t0 = jnp.where(i0 == mmv, knot, t0)
            t1 = jnp.where(idx == mmv, knot, t1)
        w_v[pl.ds(c * _L, _L)] = (tcv - t0) / (t1 - t0 + epsv)
        i0_v[pl.ds(c * _L, _L)] = i0
        i1_v[pl.ds(c * _L, _L)] = idx

    pltpu.async_copy(es_hbm.at[i0_v], e0_v, sem).wait()
    pltpu.async_copy(es_hbm.at[i1_v], e1_v, sem).wait()

    pltpu.sync_copy(w_v, w_hbm.at[pl.ds(base, chunk)])
    pltpu.sync_copy(e0_v, e0_hbm.at[pl.ds(base, chunk)])
    pltpu.sync_copy(e1_v, e1_hbm.at[pl.ds(base, chunk)])


def _bcast_body(a_ref, b_ref, w_ref, o_ref):
    a = a_ref[:, :]
    b = b_ref[:, :]
    w = w_ref[:, :]
    o_ref[:, :] = a + (b - a) * w


def kernel(t, ts, Es):
    B = t.shape[0]
    m = ts.shape[0]
    k = Es.shape[1]
    R = B * k
    chunk = B // _NW

    t1d = t.reshape(B)
    tsb = jnp.broadcast_to(ts.reshape(m, 1), (m, _L)).reshape(m * _L)
    es128 = jnp.zeros((m, 128), jnp.float32).at[:, :k].set(Es)

    sc_prep = pl.kernel(
        functools.partial(_sc_prep_body, m, k, chunk),
        out_type=(
            jax.ShapeDtypeStruct((B,), jnp.float32),
            jax.ShapeDtypeStruct((B, 128), jnp.float32),
            jax.ShapeDtypeStruct((B, 128), jnp.float32),
        ),
        mesh=plsc.VectorSubcoreMesh(core_axis_name="c", subcore_axis_name="s"),
        scratch_types=[
            pltpu.VMEM((m * _L,), jnp.float32),
            pltpu.VMEM((chunk,), jnp.float32),
            pltpu.VMEM((chunk,), jnp.float32),
            pltpu.VMEM((chunk,), jnp.int32),
            pltpu.VMEM((chunk,), jnp.int32),
            pltpu.VMEM((chunk, 128), jnp.float32),
            pltpu.VMEM((chunk, 128), jnp.float32),
            pltpu.SemaphoreType.DMA,
        ],
    )

    w1, E0, E1 = sc_prep(t1d, tsb, es128)

    w = w1.reshape(1, B)
    a = E0[:, :k].reshape(R, 1)
    b = E1[:, :k].reshape(R, 1)

    BR = 2048
    q = pl.pallas_call(
        _bcast_body,
        grid=(R // BR,),
        in_specs=[
            pl.BlockSpec((BR, 1), lambda i: (i, 0)),
            pl.BlockSpec((BR, 1), lambda i: (i, 0)),
            pl.BlockSpec((1, B), lambda i: (0, 0)),
        ],
        out_specs=pl.BlockSpec((BR, B), lambda i: (i, 0)),
        out_shape=jax.ShapeDtypeStruct((R, B), jnp.float32),
    )(a, b, w)

    return q.reshape(B, k, B).transpose(0, 2, 1)


# speedup vs baseline: 4.1688x; 4.1688x over previous
import jax
import jax.numpy as jnp
from jax.experimental import pallas as pl
from jax.experimental.pallas import tpu as pltpu


def _body(trow_ref, ts_ref, es_ref, o_ref, w_s, a_s, d_s, x_s, m_s):
    i = pl.program_id(0)
    m = ts_ref.shape[1]
    B = trow_ref.shape[1]
    k = es_ref.shape[1]
    BR = o_ref.shape[0]
    CHI = BR // k

    @pl.when(i == 0)
    def _prep():
        ts = ts_ref[:, :]
        lo = ts[0, 0]
        hi = ts[0, m - 1]
        tr = trow_ref[:, :]
        trc = jnp.clip(tr, lo, hi)
        idxc = jnp.zeros(tr.shape, jnp.int32)
        for mm in range(m):
            idxc += (ts[0, mm] < trc).astype(jnp.int32)
        idxc = jnp.clip(idxc, 1, m - 1)
        t0 = jnp.zeros(tr.shape, jnp.float32)
        t1 = jnp.zeros(tr.shape, jnp.float32)
        for mm in range(m):
            t0 = jnp.where(idxc - 1 == mm, ts[0, mm], t0)
            t1 = jnp.where(idxc == mm, ts[0, mm], t1)
        w_s[:, :] = (trc - t0) / (t1 - t0 + 1e-12)

        rows = jax.lax.broadcasted_iota(jnp.int32, (m, B), 0)
        p0 = (rows == (idxc - 1)).astype(jnp.float32)
        p1 = (rows == idxc).astype(jnp.float32)
        es = es_ref[:, :]
        dn = (((0,), (0,)), ((), ()))
        e0 = jax.lax.dot_general(p0, es, dn,
                                 preferred_element_type=jnp.float32)
        e1 = jax.lax.dot_general(p1, es, dn,
                                 preferred_element_type=jnp.float32)
        a_s[:, :] = e0
        d_s[:, :] = e1 - e0

        rr = jax.lax.broadcasted_iota(jnp.int32, (BR, CHI), 0)
        qq = jax.lax.broadcasted_iota(jnp.int32, (BR, CHI), 1)
        x_s[:, :] = (qq == rr // k).astype(jnp.float32)
        r2 = jax.lax.broadcasted_iota(jnp.int32, (BR, k), 0)
        ss = jax.lax.broadcasted_iota(jnp.int32, (BR, k), 1)
        m_s[:, :] = (ss == r2 % k).astype(jnp.float32)

    x = x_s[:, :]
    msk = m_s[:, :]
    a_blk = a_s[pl.ds(i * CHI, CHI), :]
    d_blk = d_s[pl.ds(i * CHI, CHI), :]
    ua = jnp.dot(x, a_blk, preferred_element_type=jnp.float32)
    ud = jnp.dot(x, d_blk, preferred_element_type=jnp.float32)
    a_col = jnp.sum(ua * msk, axis=1, keepdims=True)
    d_col = jnp.sum(ud * msk, axis=1, keepdims=True)
    o_ref[:, :] = a_col + d_col * w_s[:, :]


def kernel(t, ts, Es):
    B = t.shape[0]
    m = ts.shape[0]
    k = Es.shape[1]
    R = B * k

    ts2 = ts.reshape(1, m)
    trow = t.reshape(1, B)

    BR = 2048
    CHI = BR // k
    q = pl.pallas_call(
        _body,
        grid=(R // BR,),
        in_specs=[
            pl.BlockSpec((1, B), lambda i: (0, 0)),
            pl.BlockSpec((1, m), lambda i: (0, 0)),
            pl.BlockSpec((m, k), lambda i: (0, 0)),
        ],
        out_specs=pl.BlockSpec((BR, B), lambda i: (i, 0)),
        out_shape=jax.ShapeDtypeStruct((R, B), jnp.float32),
        scratch_shapes=[
            pltpu.VMEM((1, B), jnp.float32),
            pltpu.VMEM((B, k), jnp.float32),
            pltpu.VMEM((B, k), jnp.float32),
            pltpu.VMEM((BR, CHI), jnp.float32),
            pltpu.VMEM((BR, k), jnp.float32),
        ],
    )(trow, ts2, Es)

    return q.reshape(B, k, B).transpose(0, 2, 1)


# speedup vs baseline: 4.4114x; 1.0582x over previous
import jax
import jax.numpy as jnp
from jax.experimental import pallas as pl
from jax.experimental.pallas import tpu as pltpu


def _body(trow_ref, ts_ref, es_ref, o_ref, w_s, a_s, d_s, x_s, m_s):
    i = pl.program_id(0)
    m = ts_ref.shape[1]
    B = trow_ref.shape[1]
    k = es_ref.shape[1]
    BR = o_ref.shape[0]
    CHI = BR // k

    @pl.when(i == 0)
    def _prep():
        ts = ts_ref[:, :]
        lo = ts[0, 0]
        hi = ts[0, m - 1]
        tr = trow_ref[:, :]
        trc = jnp.clip(tr, lo, hi)
        idxc = jnp.zeros(tr.shape, jnp.int32)
        for mm in range(m):
            idxc += (ts[0, mm] < trc).astype(jnp.int32)
        idxc = jnp.clip(idxc, 1, m - 1)
        t0 = jnp.zeros(tr.shape, jnp.float32)
        t1 = jnp.zeros(tr.shape, jnp.float32)
        for mm in range(m):
            t0 = jnp.where(idxc - 1 == mm, ts[0, mm], t0)
            t1 = jnp.where(idxc == mm, ts[0, mm], t1)
        w_s[:, :] = (trc - t0) / (t1 - t0 + 1e-12)

        rows = jax.lax.broadcasted_iota(jnp.int32, (m, B), 0)
        p0 = (rows == (idxc - 1)).astype(jnp.float32)
        p1 = (rows == idxc).astype(jnp.float32)
        es = es_ref[:, :]
        dn = (((0,), (0,)), ((), ()))
        e0 = jax.lax.dot_general(p0, es, dn,
                                 preferred_element_type=jnp.float32)
        e1 = jax.lax.dot_general(p1, es, dn,
                                 preferred_element_type=jnp.float32)
        a_s[:, :] = e0
        d_s[:, :] = e1 - e0

        rr = jax.lax.broadcasted_iota(jnp.int32, (BR, CHI), 0)
        qq = jax.lax.broadcasted_iota(jnp.int32, (BR, CHI), 1)
        x_s[:, :] = (qq == rr // k).astype(jnp.float32)
        r2 = jax.lax.broadcasted_iota(jnp.int32, (BR, k), 0)
        ss = jax.lax.broadcasted_iota(jnp.int32, (BR, k), 1)
        m_s[:, :] = (ss == r2 % k).astype(jnp.float32)

    x = x_s[:, :]
    msk = m_s[:, :]
    a_blk = a_s[pl.ds(i * CHI, CHI), :]
    d_blk = d_s[pl.ds(i * CHI, CHI), :]
    ua = jnp.dot(x, a_blk, preferred_element_type=jnp.float32)
    ud = jnp.dot(x, d_blk, preferred_element_type=jnp.float32)
    a_col = jnp.sum(ua * msk, axis=1, keepdims=True)
    d_col = jnp.sum(ud * msk, axis=1, keepdims=True)
    o_ref[:, :] = a_col + d_col * w_s[:, :]


def kernel(t, ts, Es):
    B = t.shape[0]
    m = ts.shape[0]
    k = Es.shape[1]
    R = B * k

    ts2 = ts.reshape(1, m)
    trow = t.reshape(1, B)

    BR = 1024
    CHI = BR // k
    q = pl.pallas_call(
        _body,
        grid=(R // BR,),
        in_specs=[
            pl.BlockSpec((1, B), lambda i: (0, 0)),
            pl.BlockSpec((1, m), lambda i: (0, 0)),
            pl.BlockSpec((m, k), lambda i: (0, 0)),
        ],
        out_specs=pl.BlockSpec((BR, B), lambda i: (i, 0)),
        out_shape=jax.ShapeDtypeStruct((R, B), jnp.float32),
        scratch_shapes=[
            pltpu.VMEM((1, B), jnp.float32),
            pltpu.VMEM((B, k), jnp.float32),
            pltpu.VMEM((B, k), jnp.float32),
            pltpu.VMEM((BR, CHI), jnp.float32),
            pltpu.VMEM((BR, k), jnp.float32),
        ],
    )(trow, ts2, Es)

    return q.reshape(B, k, B).transpose(0, 2, 1)
